# baseline (speedup 1.0000x reference)
import functools

import jax
import jax.numpy as jnp
from jax import lax
from jax.experimental import pallas as pl
from jax.experimental.pallas import tpu as pltpu
from jax.experimental.pallas import tpu_sc as plsc

_L = 16


def _make_sc_kernel(V, D, B):
    NW = 32
    bpw = B // NW
    n_rows = 2 * bpw
    IDXW = 128
    HR = n_rows // 2
    n_dma = HR // IDXW
    n_grp = bpw // _L
    DC = D // _L

    mesh = plsc.VectorSubcoreMesh(core_axis_name="c", subcore_axis_name="s")

    @functools.partial(
        pl.kernel,
        out_type=jax.ShapeDtypeStruct((B,), jnp.float32),
        mesh=mesh,
        scratch_types=[
            pltpu.VMEM((n_dma, IDXW), jnp.int32),
            pltpu.VMEM((HR, 2 * D), jnp.float32),
            pltpu.VMEM((D,), jnp.float32),
            pltpu.VMEM((_L, _L), jnp.float32),
            pltpu.VMEM((bpw,), jnp.float32),
            pltpu.SemaphoreType.DMA,
        ],
        compiler_params=pltpu.CompilerParams(needs_layout_passes=False),
    )
    def run(emb_hbm, idx_hbm, r_hbm, out_hbm, idx_v, rows_v, r_v, p_v, out_v, sem):
        wid = lax.axis_index("s") * 2 + lax.axis_index("c")
        pltpu.sync_copy(r_hbm, r_v)
        r_regs = [r_v[pl.ds(c * _L, _L)] for c in range(DC)]
        iota = lax.iota(jnp.int32, _L)

        for hs in range(2):
            base = wid * n_rows + hs * HR
            for j in range(n_dma):
                pltpu.sync_copy(idx_hbm.at[pl.ds(base + j * IDXW, IDXW)], idx_v.at[j])

            copies = [
                pltpu.make_async_copy(
                    emb_hbm.at[idx_v.at[j]],
                    rows_v.at[pl.ds(j * IDXW, IDXW)],
                    sem,
                )
                for j in range(n_dma)
            ]
            for c in copies:
                c.start()
            for c in copies:
                c.wait()

            def group_body(g, carry):
                row0 = g * _L
                for j in range(_L):
                    i2 = 2 * (row0 + j)
                    acc = None
                    for c in range(DC):
                        s_c = rows_v[i2, pl.ds(c * _L, _L)]
                        o_c = rows_v[i2 + 1, pl.ds(c * _L, _L)]
                        t = (s_c * o_c) * r_regs[c]
                        acc = t if acc is None else acc + t
                    p_v[j, :] = acc
                accv = jnp.zeros((_L,), jnp.float32)
                for l in range(_L):
                    col = plsc.load_gather(p_v, [iota, jnp.full((_L,), l, jnp.int32)])
                    accv = accv + col
                sig = 1.0 / (1.0 + jnp.exp(-accv))
                out_v[pl.ds(hs * (bpw // 2) + g * _L, _L)] = sig
                return carry

            lax.fori_loop(0, n_grp // 2, group_body, 0, unroll=False)

        pltpu.sync_copy(out_v, out_hbm.at[pl.ds(wid * bpw, bpw)])

    return run


def kernel(emb, batch_ind, r):
    V, D = emb.shape
    B = batch_ind.shape[0]
    emb128 = jnp.pad(emb, ((0, 0), (0, D)))
    idx_flat = batch_ind.reshape(2 * B)
    run = _make_sc_kernel(V, D, B)
    return run(emb128, idx_flat, r)
